# speedup vs baseline: 54.6321x; 54.6321x over previous
import functools

import jax
import jax.numpy as jnp
from jax import lax
from jax.experimental import pallas as pl
from jax.experimental.pallas import tpu as pltpu

_THR = 0.3
_B = 128


def _nms_kernel(xl_ref, yl_ref, xh_ref, yh_ref, keep_ref, area_ref, *, nblk):
    B = _B
    f32 = jnp.float32

    area_ref[...] = (xh_ref[...] - xl_ref[...]) * (yh_ref[...] - yl_ref[...])

    rows_i = lax.broadcasted_iota(jnp.int32, (B, B), 0)
    cols_i = lax.broadcasted_iota(jnp.int32, (B, B), 1)
    eye = rows_i == cols_i
    tri = (rows_i < cols_i).astype(f32)

    def t_l2c(v):
        return jnp.max(jnp.where(eye, v, -1.0), axis=1, keepdims=True)

    def t_c2l(v):
        return jnp.max(jnp.where(eye, v, -1.0), axis=0, keepdims=True)

    def row(ref, j):
        return ref[pl.ds(j, 1)].reshape(1, B)

    def outer(k, _):
        cxl = t_l2c(row(xl_ref, k))
        cyl = t_l2c(row(yl_ref, k))
        cxh = t_l2c(row(xh_ref, k))
        cyh = t_l2c(row(yh_ref, k))
        car = t_l2c(row(area_ref, k))

        def cross(j, sup):
            oxl = row(xl_ref, j)
            oyl = row(yl_ref, j)
            oxh = row(xh_ref, j)
            oyh = row(yh_ref, j)
            oar = row(area_ref, j)
            okeep = row(keep_ref, j)
            xx1 = jnp.maximum(oxl, cxl)
            yy1 = jnp.maximum(oyl, cyl)
            xx2 = jnp.minimum(oxh, cxh)
            yy2 = jnp.minimum(oyh, cyh)
            w = jnp.maximum(0.0, xx2 - xx1)
            h = jnp.maximum(0.0, yy2 - yy1)
            inter = w * h
            iou = inter / (oar + car - inter + 1e-8)
            conf = (iou > _THR).astype(f32) * okeep
            return jnp.maximum(sup, jnp.max(conf, axis=1, keepdims=True))

        ext = lax.fori_loop(0, k, cross, jnp.zeros((B, 1), f32))

        oxl = row(xl_ref, k)
        oyl = row(yl_ref, k)
        oxh = row(xh_ref, k)
        oyh = row(yh_ref, k)
        oar = row(area_ref, k)
        xx1 = jnp.maximum(cxl, oxl)
        yy1 = jnp.maximum(cyl, oyl)
        xx2 = jnp.minimum(cxh, oxh)
        yy2 = jnp.minimum(cyh, oyh)
        w = jnp.maximum(0.0, xx2 - xx1)
        h = jnp.maximum(0.0, yy2 - yy1)
        inter = w * h
        iou = inter / (car + oar - inter + 1e-8)
        s_mat = (iou > _THR).astype(f32) * tri

        ext_alive_c = 1.0 - ext
        ext_alive_l = t_c2l(ext_alive_c)

        def jac_cond(st):
            _, changed, it = st
            return changed & (it < B + 2)

        def jac_body(st):
            alive_c, _, it = st
            sup_l = jnp.max(s_mat * alive_c, axis=0, keepdims=True)
            alive_l = ext_alive_l * (1.0 - sup_l)
            alive_c_new = t_l2c(alive_l)
            changed = jnp.any(alive_c_new != alive_c)
            return (alive_c_new, changed, it + jnp.int32(1))

        alive_c, _, _ = lax.while_loop(
            jac_cond, jac_body, (ext_alive_c, jnp.bool_(True), jnp.int32(0))
        )
        keep_ref[pl.ds(k, 1)] = t_c2l(alive_c).reshape(1, 1, B)
        return 0

    lax.fori_loop(0, nblk, outer, 0)


def _run_nms(xl, yl, xh, yh, nblk, interpret=False):
    kfn = functools.partial(_nms_kernel, nblk=nblk)
    return pl.pallas_call(
        kfn,
        out_shape=jax.ShapeDtypeStruct((nblk, 1, _B), jnp.float32),
        scratch_shapes=[pltpu.VMEM((nblk, 1, _B), jnp.float32)],
        interpret=interpret,
    )(xl, yl, xh, yh)


@jax.jit
def kernel(boxes, scores):
    n = boxes.shape[0]
    order = jnp.argsort(-scores)
    b = jnp.take(boxes, order, axis=0)
    s = jnp.take(scores, order)

    nblk = (n + _B - 1) // _B
    npad = nblk * _B
    pad = npad - n
    bp = jnp.pad(b, ((0, pad), (0, 0)))
    xl = bp[:, 0].reshape(nblk, 1, _B)
    yl = bp[:, 1].reshape(nblk, 1, _B)
    xh = bp[:, 2].reshape(nblk, 1, _B)
    yh = bp[:, 3].reshape(nblk, 1, _B)

    keep = _run_nms(xl, yl, xh, yh, nblk).reshape(npad)[:n]
    return jnp.concatenate([b * keep[:, None], (s * keep)[:, None]], axis=1)
